# baseline (speedup 1.0000x reference)
import functools

import jax
import jax.numpy as jnp
from jax import lax
from jax.experimental import pallas as pl
from jax.experimental.pallas import tpu as pltpu
from jax.experimental.pallas import tpu_sc as plsc

_NUM_CLASSES = 100000
_NUM_SAMPLED = 8192
_NUM_TRUE = 3
_DIM = 128
_BATCH = 1024

_BN = 2048
_CH = 2048
_NCH = _NUM_SAMPLED // _CH

_LOG_RANGE = 11.512935


def _log_expected_count(ids_f32):
    p = (jnp.log(ids_f32 + 2.0) - jnp.log(ids_f32 + 1.0)) / jnp.log(
        jnp.float32(_NUM_CLASSES + 1.0))
    return jnp.log(1.0 - jnp.exp(_NUM_SAMPLED * jnp.log(1.0 - p)))


def _proj_body(pred_ref, w_ref, b_ref, out_ref, wt_ref):
    w = w_ref[...]
    out_ref[...] = (
        jnp.dot(pred_ref[...], w, preferred_element_type=jnp.float32)
        + b_ref[...])
    wt_ref[...] = w.T


def _projection(pred, w, bias2d):
    grid = (pl.cdiv(_NUM_CLASSES, _BN),)
    return pl.pallas_call(
        _proj_body,
        grid=grid,
        in_specs=[
            pl.BlockSpec((_BATCH, _DIM), lambda k: (0, 0)),
            pl.BlockSpec((_DIM, _BN), lambda k: (0, k)),
            pl.BlockSpec((1, _BN), lambda k: (0, k)),
        ],
        out_specs=[
            pl.BlockSpec((_BATCH, _BN), lambda k: (0, k)),
            pl.BlockSpec((_BN, _DIM), lambda k: (k, 0)),
        ],
        out_shape=[
            jax.ShapeDtypeStruct((_BATCH, _NUM_CLASSES), jnp.float32),
            jax.ShapeDtypeStruct((_NUM_CLASSES, _DIM), jnp.float32),
        ],
    )(pred, w, bias2d)


@functools.lru_cache(maxsize=None)
def _make_sc_gather():
    info = plsc.get_sparse_core_info()
    nw = info.num_cores * info.num_subcores
    spw = _NUM_SAMPLED // nw
    tpw = (_BATCH * _NUM_TRUE) // nw

    @functools.partial(
        pl.kernel,
        mesh=plsc.VectorSubcoreMesh(core_axis_name="c", subcore_axis_name="s"),
        out_type=(
            jax.ShapeDtypeStruct((_NUM_SAMPLED, _DIM), jnp.float32),
            jax.ShapeDtypeStruct((_BATCH * _NUM_TRUE, _DIM), jnp.float32),
        ),
        scratch_types=[
            pltpu.VMEM((spw,), jnp.int32),
            pltpu.VMEM((tpw,), jnp.int32),
            pltpu.VMEM((spw, _DIM), jnp.float32),
            pltpu.VMEM((tpw, _DIM), jnp.float32),
            pltpu.SemaphoreType.DMA,
            pltpu.SemaphoreType.DMA,
        ],
    )
    def sc_gather(wt_hbm, sids_hbm, tids_hbm, out_s, out_t,
                  sidx_v, tidx_v, srows_v, trows_v, sem_s, sem_t):
        wid = lax.axis_index("s") * info.num_cores + lax.axis_index("c")
        sb = wid * spw
        tb = wid * tpw
        pltpu.sync_copy(sids_hbm.at[pl.ds(sb, spw)], sidx_v)
        pltpu.sync_copy(tids_hbm.at[pl.ds(tb, tpw)], tidx_v)
        cp_s = pltpu.async_copy(wt_hbm.at[sidx_v], srows_v, sem_s)
        cp_t = pltpu.async_copy(wt_hbm.at[tidx_v], trows_v, sem_t)
        cp_s.wait()
        cp_t.wait()
        pltpu.sync_copy(srows_v, out_s.at[pl.ds(sb, spw)])
        pltpu.sync_copy(trows_v, out_t.at[pl.ds(tb, tpw)])

    return sc_gather


def _loss_body(pred_ref, ws_ref, sid_ref, wt_ref, tgt_ref, out_ref, acc_ref):
    k = pl.program_id(0)
    pred = pred_ref[...]
    logits = lax.dot_general(
        pred, ws_ref[...], (((1,), (1,)), ((), ())),
        preferred_element_type=jnp.float32)
    corr = _log_expected_count(sid_ref[...].astype(jnp.float32))
    contrib = jnp.sum(jnp.exp(logits - corr), axis=1, keepdims=True)

    @pl.when(k == 0)
    def _init():
        acc_ref[...] = jnp.zeros_like(acc_ref)

    acc_ref[...] += contrib

    @pl.when(k == _NCH - 1)
    def _finish():
        wt3 = wt_ref[...].reshape(_BATCH, _NUM_TRUE, _DIM)
        true_logits = jnp.sum(pred[:, None, :] * wt3, axis=2)
        tcorr = _log_expected_count(tgt_ref[...].astype(jnp.float32))
        adj_t = true_logits - tcorr
        total = acc_ref[...] + jnp.sum(jnp.exp(adj_t), axis=1, keepdims=True)
        loss_b = jnp.log(total) - jnp.mean(adj_t, axis=1, keepdims=True)
        out_ref[...] = jnp.mean(loss_b).reshape(1, 1)


def _loss(pred, ws, sids2d, wtrue, target):
    return pl.pallas_call(
        _loss_body,
        grid=(_NCH,),
        in_specs=[
            pl.BlockSpec((_BATCH, _DIM), lambda k: (0, 0)),
            pl.BlockSpec((_CH, _DIM), lambda k: (k, 0)),
            pl.BlockSpec((1, _CH), lambda k: (0, k)),
            pl.BlockSpec((_BATCH * _NUM_TRUE, _DIM), lambda k: (0, 0)),
            pl.BlockSpec((_BATCH, _NUM_TRUE), lambda k: (0, 0)),
        ],
        out_specs=pl.BlockSpec((1, 1), lambda k: (0, 0)),
        out_shape=jax.ShapeDtypeStruct((1, 1), jnp.float32),
        scratch_shapes=[pltpu.VMEM((_BATCH, 1), jnp.float32)],
    )(pred, ws, sids2d, wtrue, target)


def kernel(pred, kernel, bias, target):
    u = jax.random.uniform(jax.random.key(42), (_NUM_SAMPLED,),
                           dtype=jnp.float32)
    sampled = jnp.clip(
        (jnp.exp(u * jnp.log(_NUM_CLASSES + 1.0)) - 1.0).astype(jnp.int32),
        0, _NUM_CLASSES - 1)

    out, wt = _projection(pred, kernel, bias.reshape(1, _NUM_CLASSES))
    ws, wtrue = _make_sc_gather()(wt, sampled, target.reshape(-1))
    loss = _loss(pred, ws, sampled.reshape(1, _NUM_SAMPLED), wtrue, target)
    return out, loss.reshape(())
